# Initial kernel scaffold; baseline (speedup 1.0000x reference)
import jax
import jax.numpy as jnp
from jax.experimental import pallas as pl


def kernel(x, E, W0, b0, W1, b1):
    raise NotImplementedError("write your pallas kernel here")


# speedup vs baseline: 1.5689x; 1.5689x over previous
import functools

import jax
import jax.numpy as jnp
from jax import lax
from jax.experimental import pallas as pl
from jax.experimental.pallas import tpu as pltpu
from jax.experimental.pallas import tpu_sc as plsc

_VOCAB = 1000000
_EMBED = 50
_WINDOW = 5
_HIDDEN = 100
_TAGS = 46
_BATCH = 16384

_NIDX = _BATCH * _WINDOW
_NC = 2
_NS = 16
_NW = _NC * _NS
_ROWS_W = _NIDX // _NW
_CHUNK = 640
_NCHUNKS = _ROWS_W // _CHUNK


def _sc_gather_body(table, idxs, out, idx_v, rows0, rows1, sem0, sem1):
    c = lax.axis_index("c")
    s = lax.axis_index("s")
    wid = s * _NC + c
    base = wid * _ROWS_W
    pltpu.sync_copy(idxs.at[pl.ds(base, _ROWS_W)], idx_v)
    bufs = (rows0, rows1)
    sems = (sem0, sem1)
    copies = [None, None]
    copies[0] = pltpu.async_copy(
        table.at[idx_v.at[pl.ds(0, _CHUNK)]], bufs[0], sems[0])
    for j in range(_NCHUNKS):
        cur = j % 2
        nxt = (j + 1) % 2
        if j + 1 < _NCHUNKS:
            copies[nxt] = pltpu.async_copy(
                table.at[idx_v.at[pl.ds((j + 1) * _CHUNK, _CHUNK)]],
                bufs[nxt], sems[nxt])
        copies[cur].wait()
        pltpu.sync_copy(bufs[cur], out.at[pl.ds(base + j * _CHUNK, _CHUNK)])


_sc_gather = pl.kernel(
    _sc_gather_body,
    out_type=jax.ShapeDtypeStruct((_NIDX, _EMBED), jnp.float32),
    scratch_types=[
        pltpu.VMEM((_ROWS_W,), jnp.int32),
        pltpu.VMEM((_CHUNK, _EMBED), jnp.float32),
        pltpu.VMEM((_CHUNK, _EMBED), jnp.float32),
        pltpu.SemaphoreType.DMA,
        pltpu.SemaphoreType.DMA,
    ],
    mesh=plsc.VectorSubcoreMesh(core_axis_name="c", subcore_axis_name="s"),
    compiler_params=pltpu.CompilerParams(use_tc_tiling_on_sc=False),
)


_BS = 2048


def _mlp_body(h_ref, w0_ref, b0_ref, w1_ref, b1_ref, out_ref):
    h = jnp.dot(h_ref[...], w0_ref[...], preferred_element_type=jnp.float32)
    h = jnp.tanh(h + b0_ref[...])
    logits = jnp.dot(h, w1_ref[...], preferred_element_type=jnp.float32)
    logits = logits + b1_ref[...]
    m = jnp.max(logits, axis=1, keepdims=True)
    sh = logits - m
    out_ref[...] = sh - jnp.log(jnp.sum(jnp.exp(sh), axis=1, keepdims=True))


@functools.partial(jax.jit, static_argnames=())
def _tc_mlp(h, w0, b0, w1, b1):
    return pl.pallas_call(
        _mlp_body,
        grid=(_BATCH // _BS,),
        in_specs=[
            pl.BlockSpec((_BS, _WINDOW * _EMBED), lambda i: (i, 0)),
            pl.BlockSpec((_WINDOW * _EMBED, _HIDDEN), lambda i: (0, 0)),
            pl.BlockSpec((1, _HIDDEN), lambda i: (0, 0)),
            pl.BlockSpec((_HIDDEN, _TAGS), lambda i: (0, 0)),
            pl.BlockSpec((1, _TAGS), lambda i: (0, 0)),
        ],
        out_specs=pl.BlockSpec((_BS, _TAGS), lambda i: (i, 0)),
        out_shape=jax.ShapeDtypeStruct((_BATCH, _TAGS), jnp.float32),
    )(h, w0, b0, w1, b1)


def kernel(x, E, W0, b0, W1, b1):
    idx = jnp.asarray(x, jnp.int32).reshape(-1)
    emb = _sc_gather(E, idx)
    h = emb.reshape(_BATCH, _WINDOW * _EMBED)
    return _tc_mlp(h, W0, b0.reshape(1, _HIDDEN), W1, b1.reshape(1, _TAGS))


# speedup vs baseline: 2.2104x; 1.4089x over previous
import jax
import jax.numpy as jnp
from jax import lax
from jax.experimental import pallas as pl
from jax.experimental.pallas import tpu as pltpu
from jax.experimental.pallas import tpu_sc as plsc

_VOCAB = 1000000
_EMBED = 50
_LANE = 128
_WINDOW = 5
_HIDDEN = 100
_TAGS = 46
_BATCH = 16384

_NIDX = _BATCH * _WINDOW
_NC = 2
_NS = 16
_NW = _NC * _NS
_ROWS_W = _NIDX // _NW
_CHUNK = 320
_NCHUNKS = _ROWS_W // _CHUNK


def _sc_gather_body(table, idxs, out, idx_v, rows0, rows1, sem0, sem1):
    c = lax.axis_index("c")
    s = lax.axis_index("s")
    wid = s * _NC + c
    base = wid * _ROWS_W
    pltpu.sync_copy(idxs.at[pl.ds(base, _ROWS_W)], idx_v)
    bufs = (rows0, rows1)
    sems = (sem0, sem1)
    copies = [None, None]
    copies[0] = pltpu.async_copy(
        table.at[idx_v.at[pl.ds(0, _CHUNK)]], bufs[0], sems[0])
    for j in range(_NCHUNKS):
        cur = j % 2
        nxt = (j + 1) % 2
        if j + 1 < _NCHUNKS:
            copies[nxt] = pltpu.async_copy(
                table.at[idx_v.at[pl.ds((j + 1) * _CHUNK, _CHUNK)]],
                bufs[nxt], sems[nxt])
        copies[cur].wait()
        pltpu.sync_copy(bufs[cur], out.at[pl.ds(base + j * _CHUNK, _CHUNK)])


_sc_gather = pl.kernel(
    _sc_gather_body,
    out_type=jax.ShapeDtypeStruct((_NIDX, _LANE), jnp.float32),
    scratch_types=[
        pltpu.VMEM((_ROWS_W,), jnp.int32),
        pltpu.VMEM((_CHUNK, _LANE), jnp.float32),
        pltpu.VMEM((_CHUNK, _LANE), jnp.float32),
        pltpu.SemaphoreType.DMA,
        pltpu.SemaphoreType.DMA,
    ],
    mesh=plsc.VectorSubcoreMesh(core_axis_name="c", subcore_axis_name="s"),
    compiler_params=pltpu.CompilerParams(use_tc_tiling_on_sc=True),
)


_BS = 2048


def _mlp_body(h_ref, w0_ref, b0_ref, w1_ref, b1_ref, out_ref):
    h = jnp.dot(h_ref[...], w0_ref[...], preferred_element_type=jnp.float32)
    h = jnp.tanh(h + b0_ref[...])
    logits = jnp.dot(h, w1_ref[...], preferred_element_type=jnp.float32)
    logits = logits + b1_ref[...]
    m = jnp.max(logits, axis=1, keepdims=True)
    sh = logits - m
    out_ref[...] = sh - jnp.log(jnp.sum(jnp.exp(sh), axis=1, keepdims=True))


def _tc_mlp(h, w0, b0, w1, b1):
    return pl.pallas_call(
        _mlp_body,
        grid=(_BATCH // _BS,),
        in_specs=[
            pl.BlockSpec((_BS, _WINDOW * _LANE), lambda i: (i, 0)),
            pl.BlockSpec((_WINDOW * _LANE, _HIDDEN), lambda i: (0, 0)),
            pl.BlockSpec((1, _HIDDEN), lambda i: (0, 0)),
            pl.BlockSpec((_HIDDEN, _TAGS), lambda i: (0, 0)),
            pl.BlockSpec((1, _TAGS), lambda i: (0, 0)),
        ],
        out_specs=pl.BlockSpec((_BS, _TAGS), lambda i: (i, 0)),
        out_shape=jax.ShapeDtypeStruct((_BATCH, _TAGS), jnp.float32),
    )(h, w0, b0, w1, b1)


def kernel(x, E, W0, b0, W1, b1):
    idx = jnp.asarray(x, jnp.int32).reshape(-1)
    Ep = jnp.pad(E, ((0, 0), (0, _LANE - _EMBED)))
    W0p = jnp.pad(W0.reshape(_WINDOW, _EMBED, _HIDDEN),
                  ((0, 0), (0, _LANE - _EMBED), (0, 0)))
    W0p = W0p.reshape(_WINDOW * _LANE, _HIDDEN)
    emb = _sc_gather(Ep, idx)
    h = emb.reshape(_BATCH, _WINDOW * _LANE)
    return _tc_mlp(h, W0p, b0.reshape(1, _HIDDEN), W1, b1.reshape(1, _TAGS))


# speedup vs baseline: 3.5477x; 1.6050x over previous
import jax
import jax.numpy as jnp
from jax import lax
from jax.experimental import pallas as pl
from jax.experimental.pallas import tpu as pltpu
from jax.experimental.pallas import tpu_sc as plsc

_VOCAB = 1000000
_EMBED = 50
_LANE = 128
_WINDOW = 5
_HIDDEN = 100
_TAGS = 46
_BATCH = 16384

_NIDX = _BATCH * _WINDOW
_NC = 2
_NS = 16
_NW = _NC * _NS
_ROWS_W = _NIDX // _NW
_CHUNK = 320
_NCHUNKS = _ROWS_W // _CHUNK


def _sc_gather_body(table, idxs, out, idx_v, rows0, rows1, sem0, sem1):
    c = lax.axis_index("c")
    s = lax.axis_index("s")
    wid = s * _NC + c
    base = wid * _ROWS_W
    pltpu.sync_copy(idxs.at[pl.ds(base, _ROWS_W)], idx_v)
    bufs = (rows0, rows1)
    sems = (sem0, sem1)
    copies = [None, None]
    copies[0] = pltpu.async_copy(
        table.at[idx_v.at[pl.ds(0, _CHUNK)]], bufs[0], sems[0])
    for j in range(_NCHUNKS):
        cur = j % 2
        nxt = (j + 1) % 2
        if j + 1 < _NCHUNKS:
            copies[nxt] = pltpu.async_copy(
                table.at[idx_v.at[pl.ds((j + 1) * _CHUNK, _CHUNK)]],
                bufs[nxt], sems[nxt])
        copies[cur].wait()
        pltpu.sync_copy(bufs[cur], out.at[pl.ds(base + j * _CHUNK, _CHUNK)])


_sc_gather = pl.kernel(
    _sc_gather_body,
    out_type=jax.ShapeDtypeStruct((_NIDX, _LANE), jnp.float32),
    scratch_types=[
        pltpu.VMEM((_ROWS_W,), jnp.int32),
        pltpu.VMEM((_CHUNK, _LANE), jnp.float32),
        pltpu.VMEM((_CHUNK, _LANE), jnp.float32),
        pltpu.SemaphoreType.DMA,
        pltpu.SemaphoreType.DMA,
    ],
    mesh=plsc.VectorSubcoreMesh(core_axis_name="c", subcore_axis_name="s"),
    compiler_params=pltpu.CompilerParams(use_tc_tiling_on_sc=True),
)


_PAD_ROWS = 20000


def _pad_body(e_ref, out_ref):
    out_ref[:, 0:_EMBED] = e_ref[...]
    out_ref[:, _EMBED:_LANE] = jnp.zeros(
        (_PAD_ROWS, _LANE - _EMBED), jnp.float32)


def _tc_pad(e):
    return pl.pallas_call(
        _pad_body,
        grid=(_VOCAB // _PAD_ROWS,),
        in_specs=[pl.BlockSpec((_PAD_ROWS, _EMBED), lambda i: (i, 0))],
        out_specs=pl.BlockSpec((_PAD_ROWS, _LANE), lambda i: (i, 0)),
        out_shape=jax.ShapeDtypeStruct((_VOCAB, _LANE), jnp.float32),
    )(e)


_BS = 2048


def _mlp_body(h_ref, w0_ref, b0_ref, w1_ref, b1_ref, out_ref):
    h = jnp.dot(h_ref[...], w0_ref[...], preferred_element_type=jnp.float32)
    h = jnp.tanh(h + b0_ref[...])
    logits = jnp.dot(h, w1_ref[...], preferred_element_type=jnp.float32)
    logits = logits + b1_ref[...]
    m = jnp.max(logits, axis=1, keepdims=True)
    sh = logits - m
    out_ref[...] = sh - jnp.log(jnp.sum(jnp.exp(sh), axis=1, keepdims=True))


def _tc_mlp(h, w0, b0, w1, b1):
    return pl.pallas_call(
        _mlp_body,
        grid=(_BATCH // _BS,),
        in_specs=[
            pl.BlockSpec((_BS, _WINDOW * _LANE), lambda i: (i, 0)),
            pl.BlockSpec((_WINDOW * _LANE, _HIDDEN), lambda i: (0, 0)),
            pl.BlockSpec((1, _HIDDEN), lambda i: (0, 0)),
            pl.BlockSpec((_HIDDEN, _TAGS), lambda i: (0, 0)),
            pl.BlockSpec((1, _TAGS), lambda i: (0, 0)),
        ],
        out_specs=pl.BlockSpec((_BS, _TAGS), lambda i: (i, 0)),
        out_shape=jax.ShapeDtypeStruct((_BATCH, _TAGS), jnp.float32),
    )(h, w0, b0, w1, b1)


def kernel(x, E, W0, b0, W1, b1):
    idx = jnp.asarray(x, jnp.int32).reshape(-1)
    Ep = _tc_pad(E)
    W0p = jnp.pad(W0.reshape(_WINDOW, _EMBED, _HIDDEN),
                  ((0, 0), (0, _LANE - _EMBED), (0, 0)))
    W0p = W0p.reshape(_WINDOW * _LANE, _HIDDEN)
    emb = _sc_gather(Ep, idx)
    h = emb.reshape(_BATCH, _WINDOW * _LANE)
    return _tc_mlp(h, W0p, b0.reshape(1, _HIDDEN), W1, b1.reshape(1, _TAGS))


# speedup vs baseline: 7.5680x; 2.1332x over previous
import jax
import jax.numpy as jnp
from jax import lax
from jax.experimental import pallas as pl
from jax.experimental.pallas import tpu as pltpu
from jax.experimental.pallas import tpu_sc as plsc

_VOCAB = 1000000
_EMBED = 50
_LANE = 128
_WINDOW = 5
_HIDDEN = 100
_TAGS = 46
_BATCH = 16384

_NIDX = _BATCH * _WINDOW
_NC = 2
_NS = 16
_NW = _NC * _NS
_ROWS_W = _NIDX // _NW
_CHUNK = 320
_NCHUNKS = _ROWS_W // _CHUNK


def _sc_gather_body(table, idxs, out, idx_v, rows0, rows1, sem0, sem1):
    c = lax.axis_index("c")
    s = lax.axis_index("s")
    wid = s * _NC + c
    base = wid * _ROWS_W
    pltpu.sync_copy(idxs.at[pl.ds(base, _ROWS_W)], idx_v)
    bufs = (rows0, rows1)
    sems = (sem0, sem1)
    copies = [None, None]
    copies[0] = pltpu.async_copy(
        table.at[idx_v.at[pl.ds(0, _CHUNK)]], bufs[0], sems[0])
    for j in range(_NCHUNKS):
        cur = j % 2
        nxt = (j + 1) % 2
        if j + 1 < _NCHUNKS:
            copies[nxt] = pltpu.async_copy(
                table.at[idx_v.at[pl.ds((j + 1) * _CHUNK, _CHUNK)]],
                bufs[nxt], sems[nxt])
        copies[cur].wait()
        pltpu.sync_copy(bufs[cur], out.at[pl.ds(base + j * _CHUNK, _CHUNK)])


_sc_gather = pl.kernel(
    _sc_gather_body,
    out_type=jax.ShapeDtypeStruct((_NIDX, _LANE), jnp.float32),
    scratch_types=[
        pltpu.VMEM((_ROWS_W,), jnp.int32),
        pltpu.VMEM((_CHUNK, _LANE), jnp.float32),
        pltpu.VMEM((_CHUNK, _LANE), jnp.float32),
        pltpu.SemaphoreType.DMA,
        pltpu.SemaphoreType.DMA,
    ],
    mesh=plsc.VectorSubcoreMesh(core_axis_name="c", subcore_axis_name="s"),
    compiler_params=pltpu.CompilerParams(use_tc_tiling_on_sc=True),
)


_PAD_ROWS = 16384


def _pad_body(et_ref, out_ref):
    t = jnp.transpose(et_ref[...], (1, 0))
    out_ref[:, 0:_EMBED] = t
    out_ref[:, _EMBED:_LANE] = jnp.zeros(
        (_PAD_ROWS, _LANE - _EMBED), jnp.float32)


def _tc_pad(et):
    return pl.pallas_call(
        _pad_body,
        grid=(pl.cdiv(_VOCAB, _PAD_ROWS),),
        in_specs=[pl.BlockSpec((_EMBED, _PAD_ROWS), lambda i: (0, i))],
        out_specs=pl.BlockSpec((_PAD_ROWS, _LANE), lambda i: (i, 0)),
        out_shape=jax.ShapeDtypeStruct((_VOCAB, _LANE), jnp.float32),
    )(et)


_BS = 2048


def _mlp_body(h_ref, w0_ref, b0_ref, w1_ref, b1_ref, out_ref):
    h = jnp.dot(h_ref[...], w0_ref[...], preferred_element_type=jnp.float32)
    h = jnp.tanh(h + b0_ref[...])
    logits = jnp.dot(h, w1_ref[...], preferred_element_type=jnp.float32)
    logits = logits + b1_ref[...]
    m = jnp.max(logits, axis=1, keepdims=True)
    sh = logits - m
    out_ref[...] = sh - jnp.log(jnp.sum(jnp.exp(sh), axis=1, keepdims=True))


def _tc_mlp(h, w0, b0, w1, b1):
    return pl.pallas_call(
        _mlp_body,
        grid=(_BATCH // _BS,),
        in_specs=[
            pl.BlockSpec((_BS, _WINDOW * _LANE), lambda i: (i, 0)),
            pl.BlockSpec((_WINDOW * _LANE, _HIDDEN), lambda i: (0, 0)),
            pl.BlockSpec((1, _HIDDEN), lambda i: (0, 0)),
            pl.BlockSpec((_HIDDEN, _TAGS), lambda i: (0, 0)),
            pl.BlockSpec((1, _TAGS), lambda i: (0, 0)),
        ],
        out_specs=pl.BlockSpec((_BS, _TAGS), lambda i: (i, 0)),
        out_shape=jax.ShapeDtypeStruct((_BATCH, _TAGS), jnp.float32),
    )(h, w0, b0, w1, b1)


def kernel(x, E, W0, b0, W1, b1):
    idx = jnp.asarray(x, jnp.int32).reshape(-1)
    Ep = _tc_pad(E.T)
    W0p = jnp.pad(W0.reshape(_WINDOW, _EMBED, _HIDDEN),
                  ((0, 0), (0, _LANE - _EMBED), (0, 0)))
    W0p = W0p.reshape(_WINDOW * _LANE, _HIDDEN)
    emb = _sc_gather(Ep, idx)
    h = emb.reshape(_BATCH, _WINDOW * _LANE)
    return _tc_mlp(h, W0p, b0.reshape(1, _HIDDEN), W1, b1.reshape(1, _TAGS))


# speedup vs baseline: 8.9067x; 1.1769x over previous
import jax
import jax.numpy as jnp
from jax import lax
from jax.experimental import pallas as pl
from jax.experimental.pallas import tpu as pltpu
from jax.experimental.pallas import tpu_sc as plsc

_VOCAB = 1000000
_EMBED = 50
_LANE = 128
_WINDOW = 5
_HIDDEN = 100
_TAGS = 46
_BATCH = 16384

_NIDX = _BATCH * _WINDOW
_NC = 2
_NS = 16
_NW = _NC * _NS
_ROWS_W = _NIDX // _NW
_CHUNK = 320
_NCHUNKS = _ROWS_W // _CHUNK


def _sc_gather_body(table, idxs, out, idx_v, rows0, rows1, sem0, sem1):
    c = lax.axis_index("c")
    s = lax.axis_index("s")
    wid = s * _NC + c
    base = wid * _ROWS_W
    pltpu.sync_copy(idxs.at[pl.ds(base, _ROWS_W)], idx_v)
    bufs = (rows0, rows1)
    sems = (sem0, sem1)
    copies = [None, None]
    copies[0] = pltpu.async_copy(
        table.at[idx_v.at[pl.ds(0, _CHUNK)]], bufs[0], sems[0])
    for j in range(_NCHUNKS):
        cur = j % 2
        nxt = (j + 1) % 2
        if j + 1 < _NCHUNKS:
            copies[nxt] = pltpu.async_copy(
                table.at[idx_v.at[pl.ds((j + 1) * _CHUNK, _CHUNK)]],
                bufs[nxt], sems[nxt])
        copies[cur].wait()
        pltpu.sync_copy(bufs[cur], out.at[pl.ds(base + j * _CHUNK, _CHUNK)])


_sc_gather = pl.kernel(
    _sc_gather_body,
    out_type=jax.ShapeDtypeStruct((_NIDX, _LANE), jnp.float32),
    scratch_types=[
        pltpu.VMEM((_ROWS_W,), jnp.int32),
        pltpu.VMEM((_CHUNK, _LANE), jnp.float32),
        pltpu.VMEM((_CHUNK, _LANE), jnp.float32),
        pltpu.SemaphoreType.DMA,
        pltpu.SemaphoreType.DMA,
    ],
    mesh=plsc.VectorSubcoreMesh(core_axis_name="c", subcore_axis_name="s"),
    compiler_params=pltpu.CompilerParams(use_tc_tiling_on_sc=True),
)


_PAD_ROWS = 16384


def _pad_body(et_ref, out_ref):
    t = jnp.transpose(et_ref[...], (1, 0))
    out_ref[:, 0:_EMBED] = t
    out_ref[:, _EMBED:_LANE] = jnp.zeros(
        (_PAD_ROWS, _LANE - _EMBED), jnp.float32)


def _tc_pad(et):
    return pl.pallas_call(
        _pad_body,
        grid=(pl.cdiv(_VOCAB, _PAD_ROWS),),
        in_specs=[pl.BlockSpec((_EMBED, _PAD_ROWS), lambda i: (0, i))],
        out_specs=pl.BlockSpec((_PAD_ROWS, _LANE), lambda i: (i, 0)),
        out_shape=jax.ShapeDtypeStruct((_VOCAB, _LANE), jnp.float32),
    )(et)


_BS = 2048


def _mlp_body(h_ref, w0_ref, b0_ref, w1_ref, b1_ref, out_ref):
    h = jnp.dot(h_ref[0], w0_ref[0], preferred_element_type=jnp.float32)
    for w in range(1, _WINDOW):
        h = h + jnp.dot(h_ref[w], w0_ref[w],
                        preferred_element_type=jnp.float32)
    h = jnp.tanh(h + b0_ref[...])
    logits = jnp.dot(h, w1_ref[...], preferred_element_type=jnp.float32)
    logits = logits + b1_ref[...]
    m = jnp.max(logits, axis=1, keepdims=True)
    sh = logits - m
    out_ref[...] = sh - jnp.log(jnp.sum(jnp.exp(sh), axis=1, keepdims=True))


def _tc_mlp(h5, w05, b0, w1, b1):
    return pl.pallas_call(
        _mlp_body,
        grid=(_BATCH // _BS,),
        in_specs=[
            pl.BlockSpec((_WINDOW, _BS, _LANE), lambda i: (0, i, 0)),
            pl.BlockSpec((_WINDOW, _LANE, _HIDDEN), lambda i: (0, 0, 0)),
            pl.BlockSpec((1, _HIDDEN), lambda i: (0, 0)),
            pl.BlockSpec((_HIDDEN, _TAGS), lambda i: (0, 0)),
            pl.BlockSpec((1, _TAGS), lambda i: (0, 0)),
        ],
        out_specs=pl.BlockSpec((_BS, _TAGS), lambda i: (i, 0)),
        out_shape=jax.ShapeDtypeStruct((_BATCH, _TAGS), jnp.float32),
    )(h5, w05, b0, w1, b1)


def kernel(x, E, W0, b0, W1, b1):
    idx = jnp.asarray(x, jnp.int32).T.reshape(-1)
    Ep = _tc_pad(E.T)
    W0p = jnp.pad(W0.reshape(_WINDOW, _EMBED, _HIDDEN),
                  ((0, 0), (0, _LANE - _EMBED), (0, 0)))
    emb = _sc_gather(Ep, idx)
    h5 = emb.reshape(_WINDOW, _BATCH, _LANE)
    return _tc_mlp(h5, W0p, b0.reshape(1, _HIDDEN), W1, b1.reshape(1, _TAGS))
